# baseline (speedup 1.0000x reference)
import functools

import jax
import jax.numpy as jnp
from jax.experimental import pallas as pl
from jax.experimental.pallas import tpu as pltpu


def _pad_hw(z, wpad):
    H, W, C = z.shape
    zrow = jnp.zeros((1, wpad, C), z.dtype)
    zl = jnp.zeros((H, 1, C), z.dtype)
    zr = jnp.zeros((H, wpad - W - 1, C), z.dtype)
    mid = jnp.concatenate([zl, z, zr], axis=1)
    return jnp.concatenate([zrow, mid, zrow], axis=0)


def _tower_kernel(x_ref, w1_ref, b1_ref, w2_ref, b2_ref, w3_ref, b3_ref,
                  o_ref, *, H, W):
    def conv(xp, wf, b, w_real, cout):
        Hp, Wp, Cin = xp.shape
        Hh = Hp - 2
        lhs = jnp.concatenate([xp[dy:dy + Hh] for dy in range(3)], axis=-1)
        p = jnp.dot(lhs.reshape(Hh * Wp, 3 * Cin), wf,
                    preferred_element_type=jnp.float32)
        p = p.reshape(Hh, Wp, 3 * cout)
        y = (p[:, 0:w_real, 0:cout]
             + p[:, 1:w_real + 1, cout:2 * cout]
             + p[:, 2:w_real + 2, 2 * cout:3 * cout])
        z = jnp.maximum(y + b, 0.0)
        z = jnp.max(z.reshape(Hh // 2, 2, w_real, cout), axis=1)
        z = jnp.max(z.reshape(Hh // 2, w_real // 2, 2, cout), axis=2)
        return z

    c1 = w1_ref.shape[1] // 3
    c2 = w2_ref.shape[1] // 3
    c3 = w3_ref.shape[1] // 3

    z = conv(x_ref[...], w1_ref[...], b1_ref[...], W, c1)
    w2r = W // 2
    z = _pad_hw(z, ((w2r + 2 + 7) // 8) * 8)
    z = conv(z, w2_ref[...], b2_ref[...], w2r, c2)
    w3r = W // 4
    z = _pad_hw(z, ((w3r + 2 + 7) // 8) * 8)
    z = conv(z, w3_ref[...], b3_ref[...], w3r, c3)

    hf, wf_ = H // 8, W // 8
    o_ref[...] = jnp.transpose(z, (2, 0, 1)).reshape(c3, hf * wf_)


def _fc_kernel(a_ref, w1_ref, b1_ref, w2_ref, o_ref):
    h = jnp.dot(a_ref[...], w1_ref[...], preferred_element_type=jnp.float32)
    h = jnp.maximum(h + b1_ref[...], 0.0)
    o_ref[...] = jnp.dot(h, w2_ref[...], preferred_element_type=jnp.float32)


def kernel(x, w1, b1, w2, b2, w3, b3, wf1, bf1, wf2, bf2):
    B, Cin, H, W = x.shape
    c1, c2, c3 = w1.shape[3], w2.shape[3], w3.shape[3]

    Wp1 = ((W + 2 + 7) // 8) * 8
    xn = jnp.transpose(x, (0, 2, 3, 1))
    xp = jnp.pad(xn, ((0, 0), (1, 1), (1, Wp1 - W - 1), (0, 0)))

    def wfold(w):
        return jnp.transpose(w, (0, 2, 1, 3)).reshape(3 * w.shape[2], 3 * w.shape[3])

    F = c3 * (H // 8) * (W // 8)
    HW = (H // 8) * (W // 8)
    tower = functools.partial(_tower_kernel, H=H, W=W)
    acts = pl.pallas_call(
        tower,
        out_shape=jax.ShapeDtypeStruct((B, c3, HW), jnp.float32),
        grid=(B,),
        in_specs=[
            pl.BlockSpec((None, H + 2, Wp1, Cin), lambda i: (i, 0, 0, 0)),
            pl.BlockSpec((3 * Cin, 3 * c1), lambda i: (0, 0)),
            pl.BlockSpec((1, c1), lambda i: (0, 0)),
            pl.BlockSpec((3 * c1, 3 * c2), lambda i: (0, 0)),
            pl.BlockSpec((1, c2), lambda i: (0, 0)),
            pl.BlockSpec((3 * c2, 3 * c3), lambda i: (0, 0)),
            pl.BlockSpec((1, c3), lambda i: (0, 0)),
        ],
        out_specs=pl.BlockSpec((None, c3, HW), lambda i: (i, 0, 0)),
        compiler_params=pltpu.CompilerParams(
            dimension_semantics=("parallel",),
            vmem_limit_bytes=64 * 1024 * 1024),
    )(xp, wfold(w1), b1.reshape(1, c1), wfold(w2), b2.reshape(1, c2),
      wfold(w3), b3.reshape(1, c3))
    acts = acts.reshape(B, F)

    NH = wf1.shape[1]
    NC = wf2.shape[1]
    NCP = ((NC + 127) // 128) * 128
    wf2p = jnp.pad(wf2, ((0, 0), (0, NCP - NC)))
    NBLK = 4
    bn = NH // NBLK
    parts = pl.pallas_call(
        _fc_kernel,
        out_shape=jax.ShapeDtypeStruct((NBLK, B, NCP), jnp.float32),
        grid=(NBLK,),
        in_specs=[
            pl.BlockSpec((B, F), lambda n: (0, 0)),
            pl.BlockSpec((F, bn), lambda n: (0, n)),
            pl.BlockSpec((1, bn), lambda n: (0, n)),
            pl.BlockSpec((bn, NCP), lambda n: (n, 0)),
        ],
        out_specs=pl.BlockSpec((None, B, NCP), lambda n: (n, 0, 0)),
        compiler_params=pltpu.CompilerParams(
            dimension_semantics=("parallel",),
            vmem_limit_bytes=64 * 1024 * 1024),
    )(acts, wf1, bf1.reshape(1, NH), wf2p)

    return parts.sum(axis=0)[:, :NC] + bf2[None, :]


# speedup vs baseline: 1.8402x; 1.8402x over previous
import functools

import jax
import jax.numpy as jnp
from jax.experimental import pallas as pl
from jax.experimental.pallas import tpu as pltpu


def _pad_hw(z, wpad):
    H, W, C = z.shape
    zrow = jnp.zeros((1, wpad, C), z.dtype)
    zl = jnp.zeros((H, 1, C), z.dtype)
    zr = jnp.zeros((H, wpad - W - 1, C), z.dtype)
    mid = jnp.concatenate([zl, z, zr], axis=1)
    return jnp.concatenate([zrow, mid, zrow], axis=0)


def _tower_kernel(x_ref, w1_ref, b1_ref, w2_ref, b2_ref, w3_ref, b3_ref,
                  o_ref, *, H, W):
    def conv_lhs(lhs, wf, b, w_real, cout):
        Hh, Wp, K = lhs.shape
        p = jnp.dot(lhs.reshape(Hh * Wp, K), wf,
                    preferred_element_type=jnp.float32)
        p = p.reshape(Hh, Wp, 3 * cout)
        y = (p[:, 0:w_real, 0:cout]
             + p[:, 1:w_real + 1, cout:2 * cout]
             + p[:, 2:w_real + 2, 2 * cout:3 * cout])
        z = jnp.maximum(y + b, 0.0)
        z = jnp.max(z.reshape(Hh // 2, 2, w_real, cout), axis=1)
        z = jnp.max(z.reshape(Hh // 2, w_real // 2, 2, cout), axis=2)
        return z

    def conv(xp, wf, b, w_real, cout):
        Hh = xp.shape[0] - 2
        lhs = jnp.concatenate([xp[dy:dy + Hh] for dy in range(3)], axis=-1)
        return conv_lhs(lhs, wf, b, w_real, cout)

    c1 = w1_ref.shape[1] // 3
    c2 = w2_ref.shape[1] // 3
    c3 = w3_ref.shape[1] // 3
    Cin = x_ref.shape[0]

    xb = x_ref[...]
    lhs1 = jnp.stack([xb[ci, dy:dy + H, :] for dy in range(3)
                      for ci in range(Cin)], axis=-1)
    z = conv_lhs(lhs1, w1_ref[...], b1_ref[...], W, c1)
    w2r = W // 2
    z = _pad_hw(z, ((w2r + 2 + 7) // 8) * 8)
    z = conv(z, w2_ref[...], b2_ref[...], w2r, c2)
    w3r = W // 4
    z = _pad_hw(z, ((w3r + 2 + 7) // 8) * 8)
    z = conv(z, w3_ref[...], b3_ref[...], w3r, c3)

    hf, wf_ = H // 8, W // 8
    o_ref[...] = jnp.transpose(z, (2, 0, 1)).reshape(c3, hf * wf_)


def _fc_kernel(a_ref, w1_ref, b1_ref, w2_ref, o_ref):
    B, C, HW = a_ref.shape
    a = a_ref[...].reshape(B, C * HW)
    h = jnp.dot(a, w1_ref[...], preferred_element_type=jnp.float32)
    h = jnp.maximum(h + b1_ref[...], 0.0)
    o_ref[...] = jnp.dot(h, w2_ref[...], preferred_element_type=jnp.float32)


def kernel(x, w1, b1, w2, b2, w3, b3, wf1, bf1, wf2, bf2):
    B, Cin, H, W = x.shape
    c1, c2, c3 = w1.shape[3], w2.shape[3], w3.shape[3]

    Wp1 = ((W + 2 + 7) // 8) * 8
    xp = jnp.pad(x, ((0, 0), (0, 0), (1, 1), (1, Wp1 - W - 1)))

    def wfold(w):
        return jnp.transpose(w, (0, 2, 1, 3)).reshape(3 * w.shape[2], 3 * w.shape[3])

    F = c3 * (H // 8) * (W // 8)
    HW = (H // 8) * (W // 8)
    tower = functools.partial(_tower_kernel, H=H, W=W)
    acts = pl.pallas_call(
        tower,
        out_shape=jax.ShapeDtypeStruct((B, c3, HW), jnp.float32),
        grid=(B,),
        in_specs=[
            pl.BlockSpec((None, Cin, H + 2, Wp1), lambda i: (i, 0, 0, 0)),
            pl.BlockSpec((3 * Cin, 3 * c1), lambda i: (0, 0)),
            pl.BlockSpec((1, c1), lambda i: (0, 0)),
            pl.BlockSpec((3 * c1, 3 * c2), lambda i: (0, 0)),
            pl.BlockSpec((1, c2), lambda i: (0, 0)),
            pl.BlockSpec((3 * c2, 3 * c3), lambda i: (0, 0)),
            pl.BlockSpec((1, c3), lambda i: (0, 0)),
        ],
        out_specs=pl.BlockSpec((None, c3, HW), lambda i: (i, 0, 0)),
        compiler_params=pltpu.CompilerParams(
            dimension_semantics=("parallel",),
            vmem_limit_bytes=64 * 1024 * 1024),
    )(xp, wfold(w1), b1.reshape(1, c1), wfold(w2), b2.reshape(1, c2),
      wfold(w3), b3.reshape(1, c3))

    NH = wf1.shape[1]
    NC = wf2.shape[1]
    NCP = ((NC + 127) // 128) * 128
    wf2p = jnp.pad(wf2, ((0, 0), (0, NCP - NC)))
    NBLK = 4
    bn = NH // NBLK
    parts = pl.pallas_call(
        _fc_kernel,
        out_shape=jax.ShapeDtypeStruct((NBLK, B, NCP), jnp.float32),
        grid=(NBLK,),
        in_specs=[
            pl.BlockSpec((B, c3, HW), lambda n: (0, 0, 0)),
            pl.BlockSpec((F, bn), lambda n: (0, n)),
            pl.BlockSpec((1, bn), lambda n: (0, n)),
            pl.BlockSpec((bn, NCP), lambda n: (n, 0)),
        ],
        out_specs=pl.BlockSpec((None, B, NCP), lambda n: (n, 0, 0)),
        compiler_params=pltpu.CompilerParams(
            dimension_semantics=("parallel",),
            vmem_limit_bytes=64 * 1024 * 1024),
    )(acts, wf1, bf1.reshape(1, NH), wf2p)

    return parts.sum(axis=0)[:, :NC] + bf2[None, :]


# speedup vs baseline: 2.0514x; 1.1148x over previous
import functools

import jax
import jax.numpy as jnp
from jax.experimental import pallas as pl
from jax.experimental.pallas import tpu as pltpu


def _pad_hw(z, wpad):
    H, W, C = z.shape
    zrow = jnp.zeros((1, wpad, C), z.dtype)
    zl = jnp.zeros((H, 1, C), z.dtype)
    zr = jnp.zeros((H, wpad - W - 1, C), z.dtype)
    mid = jnp.concatenate([zl, z, zr], axis=1)
    return jnp.concatenate([zrow, mid, zrow], axis=0)


def _tower_kernel(x_ref, w1_ref, b1_ref, w2_ref, b2_ref, w3_ref, b3_ref,
                  o_ref, *, H, W):
    def conv_lhs(lhs, wf, b, w_real, cout):
        Hh, Wp, K = lhs.shape
        p = jnp.dot(lhs.reshape(Hh * Wp, K), wf,
                    preferred_element_type=jnp.float32)
        p = p.reshape(Hh, Wp, 3 * cout)
        y = (p[:, 0:w_real, 0:cout]
             + p[:, 1:w_real + 1, cout:2 * cout]
             + p[:, 2:w_real + 2, 2 * cout:3 * cout])
        z = jnp.maximum(y + b, 0.0)
        z = jnp.max(z.reshape(Hh // 2, 2, w_real, cout), axis=1)
        z = jnp.max(z.reshape(Hh // 2, w_real // 2, 2, cout), axis=2)
        return z

    def conv(xp, wf, b, w_real, cout):
        Hh = xp.shape[0] - 2
        lhs = jnp.concatenate([xp[dy:dy + Hh] for dy in range(3)], axis=-1)
        return conv_lhs(lhs, wf, b, w_real, cout)

    c1 = w1_ref.shape[1] // 3
    c2 = w2_ref.shape[1] // 3
    c3 = w3_ref.shape[1] // 3
    Cin = x_ref.shape[0]

    xb = x_ref[...].astype(jnp.bfloat16)
    lhs1 = jnp.stack([xb[ci, dy:dy + H, :] for dy in range(3)
                      for ci in range(Cin)], axis=-1)
    z = conv_lhs(lhs1, w1_ref[...], b1_ref[...], W, c1)
    w2r = W // 2
    z = _pad_hw(z.astype(jnp.bfloat16), ((w2r + 2 + 7) // 8) * 8)
    z = conv(z, w2_ref[...], b2_ref[...], w2r, c2)
    w3r = W // 4
    z = _pad_hw(z.astype(jnp.bfloat16), ((w3r + 2 + 7) // 8) * 8)
    z = conv(z, w3_ref[...], b3_ref[...], w3r, c3)

    hf, wf_ = H // 8, W // 8
    o_ref[...] = jnp.transpose(z.astype(jnp.bfloat16), (2, 0, 1)).reshape(
        c3, hf * wf_)


def _fc_kernel(a_ref, w1_ref, b1_ref, w2_ref, o_ref):
    B, C, HW = a_ref.shape
    a = a_ref[...].reshape(B, C * HW)
    h = jnp.dot(a, w1_ref[...], preferred_element_type=jnp.float32)
    h = jnp.maximum(h + b1_ref[...], 0.0).astype(jnp.bfloat16)
    o_ref[...] = jnp.dot(h, w2_ref[...], preferred_element_type=jnp.float32)


def kernel(x, w1, b1, w2, b2, w3, b3, wf1, bf1, wf2, bf2):
    B, Cin, H, W = x.shape
    c1, c2, c3 = w1.shape[3], w2.shape[3], w3.shape[3]

    Wp1 = ((W + 2 + 7) // 8) * 8
    xp = jnp.pad(x, ((0, 0), (0, 0), (1, 1), (1, Wp1 - W - 1)))

    def wfold(w):
        return jnp.transpose(w, (0, 2, 1, 3)).reshape(
            3 * w.shape[2], 3 * w.shape[3]).astype(jnp.bfloat16)

    F = c3 * (H // 8) * (W // 8)
    HW = (H // 8) * (W // 8)
    tower = functools.partial(_tower_kernel, H=H, W=W)
    acts = pl.pallas_call(
        tower,
        out_shape=jax.ShapeDtypeStruct((B, c3, HW), jnp.bfloat16),
        grid=(B,),
        in_specs=[
            pl.BlockSpec((None, Cin, H + 2, Wp1), lambda i: (i, 0, 0, 0)),
            pl.BlockSpec((3 * Cin, 3 * c1), lambda i: (0, 0)),
            pl.BlockSpec((1, c1), lambda i: (0, 0)),
            pl.BlockSpec((3 * c1, 3 * c2), lambda i: (0, 0)),
            pl.BlockSpec((1, c2), lambda i: (0, 0)),
            pl.BlockSpec((3 * c2, 3 * c3), lambda i: (0, 0)),
            pl.BlockSpec((1, c3), lambda i: (0, 0)),
        ],
        out_specs=pl.BlockSpec((None, c3, HW), lambda i: (i, 0, 0)),
        compiler_params=pltpu.CompilerParams(
            dimension_semantics=("parallel",),
            vmem_limit_bytes=64 * 1024 * 1024),
    )(xp, wfold(w1), b1.reshape(1, c1), wfold(w2), b2.reshape(1, c2),
      wfold(w3), b3.reshape(1, c3))

    NH = wf1.shape[1]
    NC = wf2.shape[1]
    NCP = ((NC + 127) // 128) * 128
    wf2p = jnp.pad(wf2, ((0, 0), (0, NCP - NC))).astype(jnp.bfloat16)
    NBLK = 4
    bn = NH // NBLK
    parts = pl.pallas_call(
        _fc_kernel,
        out_shape=jax.ShapeDtypeStruct((NBLK, B, NCP), jnp.float32),
        grid=(NBLK,),
        in_specs=[
            pl.BlockSpec((B, c3, HW), lambda n: (0, 0, 0)),
            pl.BlockSpec((F, bn), lambda n: (0, n)),
            pl.BlockSpec((1, bn), lambda n: (0, n)),
            pl.BlockSpec((bn, NCP), lambda n: (n, 0)),
        ],
        out_specs=pl.BlockSpec((None, B, NCP), lambda n: (n, 0, 0)),
        compiler_params=pltpu.CompilerParams(
            dimension_semantics=("parallel",),
            vmem_limit_bytes=64 * 1024 * 1024),
    )(acts, wf1.astype(jnp.bfloat16), bf1.reshape(1, NH), wf2p)

    return parts.sum(axis=0)[:, :NC] + bf2[None, :]
